# baseline (speedup 1.0000x reference)
import functools

import jax
import jax.numpy as jnp
from jax import lax
from jax.experimental import pallas as pl
from jax.experimental.pallas import tpu as pltpu
from jax.experimental.pallas import tpu_sc as plsc

_info = plsc.get_sparse_core_info()
_NC, _NS, _L = _info.num_cores, _info.num_subcores, _info.num_lanes
_NW = _NC * _NS

_IDX_W = 128
_GATHERS = 5
_CHUNK = _IDX_W * _GATHERS


@functools.cache
def _build(V, D, B):
    per_w = B // _NW
    chunks = per_w // _CHUNK
    assert B % (_NW * _CHUNK) == 0 and chunks % 2 == 0, (V, D, B)
    pairs = chunks // 2
    mesh = plsc.VectorSubcoreMesh(core_axis_name="c", subcore_axis_name="s")

    @functools.partial(
        pl.kernel,
        mesh=mesh,
        out_type=jax.ShapeDtypeStruct((B, D), jnp.float32),
        scratch_types=[
            pltpu.VMEM((2 * _GATHERS, _IDX_W), jnp.int32),
            pltpu.VMEM((2, _CHUNK, D), jnp.float32),
            pltpu.SemaphoreType.DMA,
            pltpu.SemaphoreType.DMA,
        ],
        compiler_params=pltpu.CompilerParams(use_tc_tiling_on_sc=False),
    )
    def emb(table_hbm, idx_hbm, out_hbm, idx_v, rows_v, gsem, osem):
        wid = lax.axis_index("s") * _NC + lax.axis_index("c")
        idx_row0 = wid * (per_w // _IDX_W)

        def pair_body(i2, carry):
            pltpu.sync_copy(
                idx_hbm.at[pl.ds(idx_row0 + i2 * 2 * _GATHERS, 2 * _GATHERS)],
                idx_v,
            )
            for b in (0, 1):
                i = 2 * i2 + b

                @pl.when(i2 >= 1)
                def _drain():
                    pltpu.make_async_copy(
                        rows_v.at[b], out_hbm.at[pl.ds(0, _CHUNK)], osem
                    ).wait()

                copies = [
                    pltpu.make_async_copy(
                        table_hbm.at[idx_v.at[b * _GATHERS + j]],
                        rows_v.at[b, pl.ds(j * _IDX_W, _IDX_W)],
                        gsem,
                    )
                    for j in range(_GATHERS)
                ]
                for c in copies:
                    c.start()
                for c in copies:
                    c.wait()
                pltpu.make_async_copy(
                    rows_v.at[b],
                    out_hbm.at[pl.ds((idx_row0 + i * _GATHERS) * _IDX_W, _CHUNK)],
                    osem,
                ).start()
            return carry

        lax.fori_loop(0, pairs, pair_body, 0)
        pltpu.make_async_copy(rows_v.at[0], out_hbm.at[pl.ds(0, _CHUNK)], osem).wait()
        pltpu.make_async_copy(rows_v.at[1], out_hbm.at[pl.ds(0, _CHUNK)], osem).wait()

    return emb


def kernel(x, table):
    V, D = table.shape
    B = x.size
    idx = x.reshape(B // _IDX_W, _IDX_W).astype(jnp.int32)
    out = _build(V, D, B)(table, idx)
    return out.reshape(*x.shape, D)
